# baseline (speedup 1.0000x reference)
import functools

import jax
import jax.numpy as jnp
from jax import lax
from jax.experimental import pallas as pl
from jax.experimental.pallas import tpu as pltpu
from jax.experimental.pallas import tpu_sc as plsc

V = 53
D = 64
BF = 16384 * 200
NC = 2
NS = 16
NW = NC * NS
RPW = BF // NW
CH = 512
NCH = RPW // CH
NGR = CH // 16
NBUF = 2
CHW = CH * D

_mesh = plsc.VectorSubcoreMesh(core_axis_name="c", subcore_axis_name="s")


@functools.partial(
    pl.kernel,
    mesh=_mesh,
    compiler_params=pltpu.CompilerParams(
        use_tc_tiling_on_sc=False, needs_layout_passes=False
    ),
    out_type=jax.ShapeDtypeStruct((BF * D,), jnp.float32),
    scratch_types=[
        pltpu.VMEM((V * D,), jnp.float32),
        pltpu.VMEM((NBUF, CH), jnp.int32),
        pltpu.VMEM((NBUF, CHW), jnp.float32),
        pltpu.SemaphoreType.DMA,
        pltpu.SemaphoreType.DMA,
        pltpu.SemaphoreType.DMA,
        pltpu.SemaphoreType.DMA,
    ],
)
def _sc_lookup(x_hbm, tab_hbm, out_hbm, tab_v, idx_v, rows_v,
               is0, is1, os0, os1):
    idx_sem = (is0, is1)
    o_sem = (os0, os1)
    wid = lax.axis_index("s") * NC + lax.axis_index("c")
    ibase = wid * RPW
    obase = ibase * D

    pltpu.sync_copy(tab_hbm, tab_v)

    for b in range(NBUF):
        pltpu.make_async_copy(
            x_hbm.at[pl.ds(ibase + b * CH, CH)], idx_v.at[b], idx_sem[b]
        ).start()

    lane64 = lax.iota(jnp.int32, 16) * D

    def group_body(g, b):
        iv = idx_v[b, pl.ds(g * 16, 16)]
        src = iv * D
        dst = lane64 + g * (16 * D)
        for d in range(D):
            vals = plsc.load_gather(tab_v, [src + d])
            plsc.store_scatter(rows_v.at[b], [dst + d], vals)

    def chunk_body(c, b):
        @pl.when(c >= NBUF)
        def _():
            pltpu.make_async_copy(
                rows_v.at[b], out_hbm.at[pl.ds(0, CHW)], o_sem[b]
            ).wait()
        pltpu.make_async_copy(
            x_hbm.at[pl.ds(0, CH)], idx_v.at[b], idx_sem[b]
        ).wait()
        lax.fori_loop(0, NGR, lambda g, k: (group_body(g, b), k)[1], 0)
        @pl.when(c + NBUF < NCH)
        def _():
            pltpu.make_async_copy(
                x_hbm.at[pl.ds(ibase + (c + NBUF) * CH, CH)],
                idx_v.at[b],
                idx_sem[b],
            ).start()
        pltpu.make_async_copy(
            rows_v.at[b], out_hbm.at[pl.ds(obase + c * CHW, CHW)], o_sem[b]
        ).start()

    def pair_body(g2, carry):
        for b in range(NBUF):
            chunk_body(g2 * NBUF + b, b)
        return carry

    lax.fori_loop(0, NCH // NBUF, pair_body, 0)

    for b in range(NBUF):
        pltpu.make_async_copy(
            rows_v.at[b], out_hbm.at[pl.ds(0, CHW)], o_sem[b]
        ).wait()


def kernel(x, token_embedding):
    xf = x.reshape(BF).astype(jnp.int32)
    tf = token_embedding.astype(jnp.float32).reshape(V * D)
    out = _sc_lookup(xf, tf)
    return out.reshape(x.shape[0], x.shape[1], D)


# speedup vs baseline: 4.9942x; 4.9942x over previous
import functools

import jax
import jax.numpy as jnp
from jax import lax
from jax.experimental import pallas as pl
from jax.experimental.pallas import tpu as pltpu
from jax.experimental.pallas import tpu_sc as plsc

V = 53
D = 64
B = 16384
S = 200
NC = 2
NS = 16
NW = NC * NS
BTPW = 4
BPW = BTPW * 128
NBUF = 2
CHW = 8 * BTPW * 8 * 128

_mesh = plsc.VectorSubcoreMesh(core_axis_name="c", subcore_axis_name="s")


@functools.partial(
    pl.kernel,
    mesh=_mesh,
    compiler_params=pltpu.CompilerParams(
        use_tc_tiling_on_sc=False, needs_layout_passes=False
    ),
    out_type=jax.ShapeDtypeStruct((S, 8, 128, 8, 128), jnp.float32),
    scratch_types=[
        pltpu.VMEM((V * D,), jnp.float32),
        pltpu.VMEM((NBUF, BPW), jnp.int32),
        pltpu.VMEM((NBUF, 1, 8, BTPW, 8, 128), jnp.float32),
        pltpu.SemaphoreType.DMA,
        pltpu.SemaphoreType.DMA,
        pltpu.SemaphoreType.DMA,
        pltpu.SemaphoreType.DMA,
    ],
)
def _sc_lookup_t(xt_hbm, tab_hbm, out_hbm, tab_v, idx_v, rows_v,
                 is0, is1, os0, os1):
    idx_sem = (is0, is1)
    o_sem = (os0, os1)
    wid = lax.axis_index("s") * NC + lax.axis_index("c")
    bbase = wid * BPW
    btbase = wid * BTPW

    pltpu.sync_copy(tab_hbm, tab_v)

    for b in range(NBUF):
        pltpu.make_async_copy(
            xt_hbm.at[pl.ds(b * B + bbase, BPW)], idx_v.at[b], idx_sem[b]
        ).start()

    def chunk_body(s, b):
        @pl.when(s >= NBUF)
        def _():
            pltpu.make_async_copy(
                rows_v.at[b],
                out_hbm.at[pl.ds(0, 1), :, pl.ds(btbase, BTPW)],
                o_sem[b],
            ).wait()
        pltpu.make_async_copy(
            xt_hbm.at[pl.ds(0, BPW)], idx_v.at[b], idx_sem[b]
        ).wait()

        for bt in range(BTPW):
            def g_body(g, carry):
                iv = idx_v[b, pl.ds(bt * 128 + g * 16, 16)]
                for dt in range(8):
                    for dr in range(8):
                        vals = plsc.load_gather(tab_v, [iv + (dt * 8 + dr) * V])
                        rows_v[b, 0, dt, bt, dr, pl.ds(g * 16, 16)] = vals
                return carry
            lax.fori_loop(0, 8, g_body, 0)

        @pl.when(s + NBUF < S)
        def _():
            pltpu.make_async_copy(
                xt_hbm.at[pl.ds((s + NBUF) * B + bbase, BPW)],
                idx_v.at[b],
                idx_sem[b],
            ).start()
        pltpu.make_async_copy(
            rows_v.at[b],
            out_hbm.at[pl.ds(s, 1), :, pl.ds(btbase, BTPW)],
            o_sem[b],
        ).start()

    def pair_body(g2, carry):
        for b in range(NBUF):
            chunk_body(g2 * NBUF + b, b)
        return carry

    lax.fori_loop(0, S // NBUF, pair_body, 0)

    for b in range(NBUF):
        pltpu.make_async_copy(
            rows_v.at[b],
            out_hbm.at[pl.ds(0, 1), :, pl.ds(btbase, BTPW)],
            o_sem[b],
        ).wait()


def kernel(x, token_embedding):
    xt = jnp.transpose(x).reshape(S * B).astype(jnp.int32)
    tabt = jnp.transpose(token_embedding.astype(jnp.float32)).reshape(D * V)
    out5 = _sc_lookup_t(xt, tabt)
    out = jnp.transpose(out5, (2, 4, 0, 1, 3)).reshape(B, S, D)
    return out
